# Initial kernel scaffold; baseline (speedup 1.0000x reference)
import jax
import jax.numpy as jnp
from jax.experimental import pallas as pl


def kernel(x, W):
    raise NotImplementedError("write your pallas kernel here")


# speedup vs baseline: 22.9186x; 22.9186x over previous
import functools

import jax
import jax.numpy as jnp
from jax import lax
from jax.experimental import pallas as pl
from jax.experimental.pallas import tpu as pltpu
from jax.experimental.pallas import tpu_sc as plsc

VOCAB = 1000000
D = 64
B = 4096
L = 200
N = B * L

NC = 2
NS = 16
NW = NC * NS
PER_W = N // NW

CH = 128
KCH = 8
ROWS = CH * KCH
NCHUNK = PER_W // CH
NOUTER = PER_W // ROWS


def _emb_body(x_hbm, w_hbm, out_hbm, idx_v, rows_v, sem):
    wid = lax.axis_index("s") * NC + lax.axis_index("c")
    pltpu.sync_copy(x_hbm.at[wid], idx_v)
    base = wid * PER_W

    def outer(o, carry):
        copies = []
        for k in range(KCH):
            j = o * KCH + k
            copies.append(
                pltpu.async_copy(
                    w_hbm.at[idx_v.at[j]],
                    rows_v.at[pl.ds(k * CH, CH)],
                    sem,
                )
            )
        for c in copies:
            c.wait()
        pltpu.sync_copy(rows_v, out_hbm.at[pl.ds(base + o * ROWS, ROWS)])
        return carry

    lax.fori_loop(0, NOUTER, outer, 0)


@jax.jit
def _embedding_lookup(x_flat, W):
    f = functools.partial(
        pl.kernel,
        mesh=plsc.VectorSubcoreMesh(core_axis_name="c", subcore_axis_name="s"),
        out_type=jax.ShapeDtypeStruct((N, D), jnp.float32),
        scratch_types=[
            pltpu.VMEM((NCHUNK, CH), jnp.int32),
            pltpu.VMEM((ROWS, D), jnp.float32),
            pltpu.SemaphoreType.DMA,
        ],
        compiler_params=pltpu.CompilerParams(use_tc_tiling_on_sc=False),
    )(_emb_body)
    return f(x_flat.reshape(NW, NCHUNK, CH), W)


def kernel(x, W):
    out_flat = _embedding_lookup(x.reshape(-1), W)
    return out_flat.reshape(B, L, D)


# speedup vs baseline: 23.0728x; 1.0067x over previous
import functools

import jax
import jax.numpy as jnp
from jax import lax
from jax.experimental import pallas as pl
from jax.experimental.pallas import tpu as pltpu
from jax.experimental.pallas import tpu_sc as plsc

VOCAB = 1000000
D = 64
B = 4096
L = 200
N = B * L

NC = 2
NS = 16
NW = NC * NS
PER_W = N // NW

CH = 128
KCH = 4
ROWS = CH * KCH
NCHUNK = PER_W // CH
NOUTER = PER_W // ROWS
NPAIR = NOUTER // 2


def _emb_body(x_hbm, w_hbm, out_hbm, idx_v, rows0, rows1, gsem0, gsem1,
              wsem0, wsem1):
    wid = lax.axis_index("s") * NC + lax.axis_index("c")
    pltpu.sync_copy(x_hbm.at[wid], idx_v)
    base = wid * PER_W

    rows = (rows0, rows1)
    gsem = (gsem0, gsem1)
    wsem = (wsem0, wsem1)

    def fire_gathers(o, buf):
        for k in range(KCH):
            pltpu.async_copy(
                w_hbm.at[idx_v.at[o * KCH + k]],
                rows[buf].at[pl.ds(k * CH, CH)],
                gsem[buf],
            )

    def drain_gathers(buf):
        for k in range(KCH):
            pltpu.make_async_copy(
                w_hbm.at[idx_v.at[0]],
                rows[buf].at[pl.ds(k * CH, CH)],
                gsem[buf],
            ).wait()

    def fire_write(o, buf):
        pltpu.async_copy(
            rows[buf], out_hbm.at[pl.ds(base + o * ROWS, ROWS)], wsem[buf]
        )

    def drain_write(buf):
        pltpu.make_async_copy(
            rows[buf], out_hbm.at[pl.ds(base, ROWS)], wsem[buf]
        ).wait()

    fire_gathers(0, 0)

    def step(i, carry):
        o = 2 * i
        drain_gathers(0)
        fire_write(o, 0)

        @pl.when(i > 0)
        def _():
            drain_write(1)
        fire_gathers(o + 1, 1)
        drain_gathers(1)
        fire_write(o + 1, 1)
        drain_write(0)

        @pl.when(o + 2 < NOUTER)
        def _():
            fire_gathers(o + 2, 0)
        return carry

    lax.fori_loop(0, NPAIR, step, 0)
    drain_write(1)


@jax.jit
def _embedding_lookup(x_flat, W):
    f = functools.partial(
        pl.kernel,
        mesh=plsc.VectorSubcoreMesh(core_axis_name="c", subcore_axis_name="s"),
        out_type=jax.ShapeDtypeStruct((N, D), jnp.float32),
        scratch_types=[
            pltpu.VMEM((NCHUNK, CH), jnp.int32),
            pltpu.VMEM((ROWS, D), jnp.float32),
            pltpu.VMEM((ROWS, D), jnp.float32),
            pltpu.SemaphoreType.DMA,
            pltpu.SemaphoreType.DMA,
            pltpu.SemaphoreType.DMA,
            pltpu.SemaphoreType.DMA,
        ],
        compiler_params=pltpu.CompilerParams(use_tc_tiling_on_sc=False),
    )(_emb_body)
    return f(x_flat.reshape(NW, NCHUNK, CH), W)


def kernel(x, W):
    out_flat = _embedding_lookup(x.reshape(-1), W)
    return out_flat.reshape(B, L, D)
